# Initial kernel scaffold; baseline (speedup 1.0000x reference)
import jax
import jax.numpy as jnp
from jax.experimental import pallas as pl


def kernel(x, edge_index, edge_weight, batch, W1, b1, W2, b2, Wfc, bfc):
    raise NotImplementedError("write your pallas kernel here")


# speedup vs baseline: 34.7300x; 34.7300x over previous
import functools

import jax
import jax.numpy as jnp
from jax import lax
from jax.experimental import pallas as pl
from jax.experimental.pallas import tpu as pltpu
from jax.experimental.pallas import tpu_sc as plsc

N_NODES = 10000
N_PAD = 10240
N_EDGES = 320000
NW = 32
NSEG = 80
SEG = 128
EP = NW * NSEG * SEG
ROWS_PER_TILE = N_PAD // 16
IN_CH = 128
HID = 16
N_GRAPHS = 64

_mesh = plsc.VectorSubcoreMesh(core_axis_name="c", subcore_axis_name="s")
_sc_params = pltpu.CompilerParams(use_tc_tiling_on_sc=False)


@functools.partial(
    pl.kernel,
    out_type=jax.ShapeDtypeStruct((2, N_PAD), jnp.float32),
    mesh=_mesh,
    compiler_params=_sc_params,
    scratch_types=[
        pltpu.VMEM((NSEG, SEG), jnp.int32),
        pltpu.VMEM((NSEG, SEG), jnp.float32),
        pltpu.VMEM((SEG,), jnp.float32),
        pltpu.VMEM_SHARED((N_PAD,), jnp.float32),
    ],
)
def _deg_sc(dst_hbm, ew_hbm, out_hbm, dstv, ewv, zbuf, deg_sh):
    c = lax.axis_index("c")
    s = lax.axis_index("s")
    w = c * 16 + s
    pltpu.sync_copy(dst_hbm.at[w], dstv)
    pltpu.sync_copy(ew_hbm.at[w], ewv)

    def _zv(i, carry):
        zbuf[pl.ds(i * 16, 16)] = jnp.zeros((16,), jnp.float32)
        return carry

    lax.fori_loop(0, SEG // 16, _zv, 0)

    def _zc(j, carry):
        pltpu.sync_copy(zbuf, deg_sh.at[pl.ds(s * ROWS_PER_TILE + j * SEG, SEG)])
        return carry

    lax.fori_loop(0, ROWS_PER_TILE // SEG, _zc, 0)
    plsc.subcore_barrier()

    def _seg(i, carry):
        pltpu.sync_copy(ewv.at[i], deg_sh.at[dstv.at[i]], add=True)
        return carry

    lax.fori_loop(0, NSEG, _seg, 0)
    plsc.subcore_barrier()
    pltpu.sync_copy(
        deg_sh.at[pl.ds(s * ROWS_PER_TILE, ROWS_PER_TILE)],
        out_hbm.at[c, pl.ds(s * ROWS_PER_TILE, ROWS_PER_TILE)],
    )


@functools.partial(
    pl.kernel,
    out_type=jax.ShapeDtypeStruct((2, N_PAD, HID), jnp.float32),
    mesh=_mesh,
    compiler_params=_sc_params,
    scratch_types=[
        pltpu.VMEM((NSEG, SEG), jnp.int32),
        pltpu.VMEM((NSEG, SEG), jnp.int32),
        pltpu.VMEM((NSEG, SEG), jnp.float32),
        pltpu.VMEM((SEG, HID), jnp.float32),
        pltpu.VMEM_SHARED((N_PAD, HID), jnp.float32),
    ],
)
def _agg_sc(hh_hbm, src_hbm, dst_hbm, ew_hbm, out_hbm, srcv, dstv, ewv, rows, agg_sh):
    c = lax.axis_index("c")
    s = lax.axis_index("s")
    w = c * 16 + s
    pltpu.sync_copy(src_hbm.at[w], srcv)
    pltpu.sync_copy(dst_hbm.at[w], dstv)
    pltpu.sync_copy(ew_hbm.at[w], ewv)

    def _zv(i, carry):
        rows[i] = jnp.zeros((HID,), jnp.float32)
        return carry

    lax.fori_loop(0, SEG, _zv, 0)

    def _zc(j, carry):
        pltpu.sync_copy(rows, agg_sh.at[pl.ds(s * ROWS_PER_TILE + j * SEG, SEG)])
        return carry

    lax.fori_loop(0, ROWS_PER_TILE // SEG, _zc, 0)
    plsc.subcore_barrier()

    def _seg(i, carry):
        pltpu.sync_copy(hh_hbm.at[srcv.at[i]], rows)

        def _blk(b, carry2):
            ew16 = ewv[i, pl.ds(b * 16, 16)]
            for j in range(16):
                e = b * 16 + j
                rows[e] = rows[e] * ew16[j]
            return carry2

        lax.fori_loop(0, SEG // 16, _blk, 0)
        pltpu.sync_copy(rows, agg_sh.at[dstv.at[i]], add=True)
        return carry

    lax.fori_loop(0, NSEG, _seg, 0)
    plsc.subcore_barrier()
    pltpu.sync_copy(
        agg_sh.at[pl.ds(s * ROWS_PER_TILE, ROWS_PER_TILE)],
        out_hbm.at[c, pl.ds(s * ROWS_PER_TILE, ROWS_PER_TILE)],
    )


def _pre_tc_body(x_ref, w1_ref, degp_ref, hh_ref, dinv_ref):
    deg = degp_ref[0, :N_NODES] + degp_ref[1, :N_NODES] + 1.0
    dinv = jnp.where(deg > 0, lax.rsqrt(deg), 0.0)
    h = jnp.dot(x_ref[...], w1_ref[...], preferred_element_type=jnp.float32)
    hh_ref[:N_NODES, :] = h * dinv[:, None]
    hh_ref[N_NODES:, :] = jnp.zeros((N_PAD - N_NODES, HID), jnp.float32)
    dinv_ref[...] = dinv[:, None]


_pre_tc = pl.pallas_call(
    _pre_tc_body,
    out_shape=(
        jax.ShapeDtypeStruct((N_PAD, HID), jnp.float32),
        jax.ShapeDtypeStruct((N_NODES, 1), jnp.float32),
    ),
)


def _mid_tc_body(aggp_ref, hh1_ref, dinv_ref, b1_ref, w2_ref, hh2_ref):
    dinv = dinv_ref[...]
    a = aggp_ref[0, :N_NODES, :] + aggp_ref[1, :N_NODES, :] + hh1_ref[:N_NODES, :]
    h1 = jnp.maximum(dinv * a + b1_ref[...][None, :], 0.0)
    h1w = jnp.dot(h1, w2_ref[...], preferred_element_type=jnp.float32)
    hh2_ref[:N_NODES, :] = h1w * dinv
    hh2_ref[N_NODES:, :] = jnp.zeros((N_PAD - N_NODES, HID), jnp.float32)


_mid_tc = pl.pallas_call(
    _mid_tc_body,
    out_shape=jax.ShapeDtypeStruct((N_PAD, HID), jnp.float32),
)


def _post_tc_body(aggp_ref, hh2_ref, dinv_ref, b2_ref, batch_ref, wfc_ref,
                  bfc_ref, out_ref):
    dinv = dinv_ref[...]
    a = aggp_ref[0, :N_NODES, :] + aggp_ref[1, :N_NODES, :] + hh2_ref[:N_NODES, :]
    h2 = dinv * a + b2_ref[...][None, :]
    b = batch_ref[...]
    gids = lax.broadcasted_iota(jnp.int32, (N_NODES, N_GRAPHS), 1)
    m = (b[:, None] == gids).astype(jnp.float32)
    sums = lax.dot_general(m, h2, (((0,), (0,)), ((), ())),
                           preferred_element_type=jnp.float32)
    counts = jnp.sum(m, axis=0)[:, None]
    pooled = sums / jnp.maximum(counts, 1.0)
    out_ref[...] = (
        jnp.dot(pooled, wfc_ref[...], preferred_element_type=jnp.float32)
        + bfc_ref[...][None, :]
    )


_post_tc = pl.pallas_call(
    _post_tc_body,
    out_shape=jax.ShapeDtypeStruct((N_GRAPHS, 1), jnp.float32),
)


def kernel(x, edge_index, edge_weight, batch, W1, b1, W2, b2, Wfc, bfc):
    src = edge_index[0]
    dst = edge_index[1]
    pad = EP - N_EDGES
    padidx = N_NODES + (jnp.arange(pad, dtype=jnp.int32) % (N_PAD - N_NODES))
    srcp = jnp.concatenate([src, padidx]).reshape(NW, NSEG, SEG)
    dstp = jnp.concatenate([dst, padidx]).reshape(NW, NSEG, SEG)
    ewp = jnp.concatenate(
        [edge_weight, jnp.zeros((pad,), jnp.float32)]
    ).reshape(NW, NSEG, SEG)

    degp = _deg_sc(dstp, ewp)
    hh1, dinv = _pre_tc(x, W1, degp)
    agg1 = _agg_sc(hh1, srcp, dstp, ewp)
    hh2 = _mid_tc(agg1, hh1, dinv, b1, W2)
    agg2 = _agg_sc(hh2, srcp, dstp, ewp)
    return _post_tc(agg2, hh2, dinv, b2, batch, Wfc, bfc)


# speedup vs baseline: 53.2734x; 1.5339x over previous
import functools

import jax
import jax.numpy as jnp
from jax import lax
from jax.experimental import pallas as pl
from jax.experimental.pallas import tpu as pltpu
from jax.experimental.pallas import tpu_sc as plsc

N_NODES = 10000
N_PAD = 10240
N_EDGES = 320000
NW = 32
NSEG = 80
SEG = 128
EP = NW * NSEG * SEG
ROWS_PER_TILE = N_PAD // 16
IN_CH = 128
HID = 16
N_GRAPHS = 64

_mesh = plsc.VectorSubcoreMesh(core_axis_name="c", subcore_axis_name="s")
_sc_params = pltpu.CompilerParams(use_tc_tiling_on_sc=False)


@functools.partial(
    pl.kernel,
    out_type=jax.ShapeDtypeStruct((2, N_PAD), jnp.float32),
    mesh=_mesh,
    compiler_params=_sc_params,
    scratch_types=[
        pltpu.VMEM((NSEG, SEG), jnp.int32),
        pltpu.VMEM((NSEG, SEG), jnp.float32),
        pltpu.VMEM((SEG,), jnp.float32),
        pltpu.VMEM_SHARED((N_PAD,), jnp.float32),
    ],
)
def _deg_sc(dst_hbm, ew_hbm, out_hbm, dstv, ewv, zbuf, deg_sh):
    c = lax.axis_index("c")
    s = lax.axis_index("s")
    w = c * 16 + s
    pltpu.sync_copy(dst_hbm.at[w], dstv)
    pltpu.sync_copy(ew_hbm.at[w], ewv)

    def _zv(i, carry):
        zbuf[pl.ds(i * 16, 16)] = jnp.zeros((16,), jnp.float32)
        return carry

    lax.fori_loop(0, SEG // 16, _zv, 0)

    def _zc(j, carry):
        pltpu.sync_copy(zbuf, deg_sh.at[pl.ds(s * ROWS_PER_TILE + j * SEG, SEG)])
        return carry

    lax.fori_loop(0, ROWS_PER_TILE // SEG, _zc, 0)
    plsc.subcore_barrier()

    def _seg(i, carry):
        pltpu.sync_copy(ewv.at[i], deg_sh.at[dstv.at[i]], add=True)
        return carry

    lax.fori_loop(0, NSEG, _seg, 0)
    plsc.subcore_barrier()
    pltpu.sync_copy(
        deg_sh.at[pl.ds(s * ROWS_PER_TILE, ROWS_PER_TILE)],
        out_hbm.at[c, pl.ds(s * ROWS_PER_TILE, ROWS_PER_TILE)],
    )


@functools.partial(
    pl.kernel,
    out_type=jax.ShapeDtypeStruct((2, N_PAD, HID), jnp.float32),
    mesh=_mesh,
    compiler_params=_sc_params,
    scratch_types=[
        pltpu.VMEM((NSEG, SEG), jnp.int32),
        pltpu.VMEM((NSEG, SEG), jnp.int32),
        pltpu.VMEM((NSEG, SEG), jnp.float32),
        pltpu.VMEM((4, SEG, HID), jnp.float32),
        pltpu.VMEM_SHARED((N_PAD, HID), jnp.float32),
        pltpu.SemaphoreType.DMA,
        pltpu.SemaphoreType.DMA,
        pltpu.SemaphoreType.DMA,
        pltpu.SemaphoreType.DMA,
        pltpu.SemaphoreType.DMA,
    ],
)
def _agg_sc(hh_hbm, src_hbm, dst_hbm, ew_hbm, out_hbm, srcv, dstv, ewv,
            rows, agg_sh, gsem0, gsem1, gsem2, gsem3, ssem):
    gsems = (gsem0, gsem1, gsem2, gsem3)
    c = lax.axis_index("c")
    s = lax.axis_index("s")
    w = c * 16 + s
    pltpu.sync_copy(src_hbm.at[w], srcv)
    pltpu.sync_copy(dst_hbm.at[w], dstv)
    pltpu.sync_copy(ew_hbm.at[w], ewv)

    def _zv(i, carry):
        rows[0, i] = jnp.zeros((HID,), jnp.float32)
        return carry

    lax.fori_loop(0, SEG, _zv, 0)

    def _zc(j, carry):
        pltpu.sync_copy(rows.at[0],
                        agg_sh.at[pl.ds(s * ROWS_PER_TILE + j * SEG, SEG)])
        return carry

    lax.fori_loop(0, ROWS_PER_TILE // SEG, _zc, 0)
    plsc.subcore_barrier()

    def _body(k, carry):
        t0 = 4 * k
        gathers = [
            pltpu.async_copy(hh_hbm.at[srcv.at[t0 + q]], rows.at[q], gsems[q])
            for q in range(4)
        ]
        scatters = []
        for q in range(4):
            t = t0 + q
            gathers[q].wait()

            def _blk(bb, carry2, _q=q, _t=t):
                ew16 = ewv[_t, pl.ds(bb * 16, 16)]
                for j in range(16):
                    e = bb * 16 + j
                    rows[_q, e] = rows[_q, e] * ew16[j]
                return carry2

            lax.fori_loop(0, SEG // 16, _blk, 0)
            scatters.append(
                pltpu.async_copy(rows.at[q], agg_sh.at[dstv.at[t]], ssem,
                                 add=True))
        for sc in scatters:
            sc.wait()
        return carry

    lax.fori_loop(0, NSEG // 4, _body, 0)
    plsc.subcore_barrier()
    pltpu.sync_copy(
        agg_sh.at[pl.ds(s * ROWS_PER_TILE, ROWS_PER_TILE)],
        out_hbm.at[c, pl.ds(s * ROWS_PER_TILE, ROWS_PER_TILE)],
    )


def _pre_tc_body(x_ref, w1_ref, degp_ref, hh_ref, dinv_ref):
    deg = degp_ref[0, :N_NODES] + degp_ref[1, :N_NODES] + 1.0
    dinv = jnp.where(deg > 0, lax.rsqrt(deg), 0.0)
    h = jnp.dot(x_ref[...], w1_ref[...], preferred_element_type=jnp.float32)
    hh_ref[:N_NODES, :] = h * dinv[:, None]
    hh_ref[N_NODES:, :] = jnp.zeros((N_PAD - N_NODES, HID), jnp.float32)
    dinv_ref[...] = dinv[:, None]


_pre_tc = pl.pallas_call(
    _pre_tc_body,
    out_shape=(
        jax.ShapeDtypeStruct((N_PAD, HID), jnp.float32),
        jax.ShapeDtypeStruct((N_NODES, 1), jnp.float32),
    ),
)


def _mid_tc_body(aggp_ref, hh1_ref, dinv_ref, b1_ref, w2_ref, hh2_ref):
    dinv = dinv_ref[...]
    a = aggp_ref[0, :N_NODES, :] + aggp_ref[1, :N_NODES, :] + hh1_ref[:N_NODES, :]
    h1 = jnp.maximum(dinv * a + b1_ref[...][None, :], 0.0)
    h1w = jnp.dot(h1, w2_ref[...], preferred_element_type=jnp.float32)
    hh2_ref[:N_NODES, :] = h1w * dinv
    hh2_ref[N_NODES:, :] = jnp.zeros((N_PAD - N_NODES, HID), jnp.float32)


_mid_tc = pl.pallas_call(
    _mid_tc_body,
    out_shape=jax.ShapeDtypeStruct((N_PAD, HID), jnp.float32),
)


def _post_tc_body(aggp_ref, hh2_ref, dinv_ref, b2_ref, batch_ref, wfc_ref,
                  bfc_ref, out_ref):
    dinv = dinv_ref[...]
    a = aggp_ref[0, :N_NODES, :] + aggp_ref[1, :N_NODES, :] + hh2_ref[:N_NODES, :]
    h2 = dinv * a + b2_ref[...][None, :]
    b = batch_ref[...]
    gids = lax.broadcasted_iota(jnp.int32, (N_NODES, N_GRAPHS), 1)
    m = (b[:, None] == gids).astype(jnp.float32)
    sums = lax.dot_general(m, h2, (((0,), (0,)), ((), ())),
                           preferred_element_type=jnp.float32)
    counts = jnp.sum(m, axis=0)[:, None]
    pooled = sums / jnp.maximum(counts, 1.0)
    out_ref[...] = (
        jnp.dot(pooled, wfc_ref[...], preferred_element_type=jnp.float32)
        + bfc_ref[...][None, :]
    )


_post_tc = pl.pallas_call(
    _post_tc_body,
    out_shape=jax.ShapeDtypeStruct((N_GRAPHS, 1), jnp.float32),
)


def kernel(x, edge_index, edge_weight, batch, W1, b1, W2, b2, Wfc, bfc):
    src = edge_index[0]
    dst = edge_index[1]
    pad = EP - N_EDGES
    padidx = N_NODES + (jnp.arange(pad, dtype=jnp.int32) % (N_PAD - N_NODES))
    srcp = jnp.concatenate([src, padidx]).reshape(NW, NSEG, SEG)
    dstp = jnp.concatenate([dst, padidx]).reshape(NW, NSEG, SEG)
    ewp = jnp.concatenate(
        [edge_weight, jnp.zeros((pad,), jnp.float32)]
    ).reshape(NW, NSEG, SEG)

    degp = _deg_sc(dstp, ewp)
    hh1, dinv = _pre_tc(x, W1, degp)
    agg1 = _agg_sc(hh1, srcp, dstp, ewp)
    hh2 = _mid_tc(agg1, hh1, dinv, b1, W2)
    agg2 = _agg_sc(hh2, srcp, dstp, ewp)
    return _post_tc(agg2, hh2, dinv, b2, batch, Wfc, bfc)


# speedup vs baseline: 54.2280x; 1.0179x over previous
import functools

import jax
import jax.numpy as jnp
from jax import lax
from jax.experimental import pallas as pl
from jax.experimental.pallas import tpu as pltpu
from jax.experimental.pallas import tpu_sc as plsc

N_NODES = 10000
N_PAD = 10240
N_EDGES = 320000
NW = 32
NSEG = 80
SEG = 128
EP = NW * NSEG * SEG
ROWS_PER_TILE = N_PAD // 16
IN_CH = 128
HID = 16
N_GRAPHS = 64

_mesh = plsc.VectorSubcoreMesh(core_axis_name="c", subcore_axis_name="s")
_sc_params = pltpu.CompilerParams(use_tc_tiling_on_sc=False)


@functools.partial(
    pl.kernel,
    out_type=jax.ShapeDtypeStruct((2, N_PAD), jnp.float32),
    mesh=_mesh,
    compiler_params=_sc_params,
    scratch_types=[
        pltpu.VMEM((NSEG, SEG), jnp.int32),
        pltpu.VMEM((NSEG, SEG), jnp.float32),
        pltpu.VMEM((SEG,), jnp.float32),
        pltpu.VMEM_SHARED((N_PAD,), jnp.float32),
        pltpu.SemaphoreType.DMA,
    ],
)
def _deg_sc(dst_hbm, ew_hbm, out_hbm, dstv, ewv, zbuf, deg_sh, ssem):
    c = lax.axis_index("c")
    s = lax.axis_index("s")
    w = c * 16 + s
    pltpu.sync_copy(dst_hbm.at[w], dstv)
    pltpu.sync_copy(ew_hbm.at[w], ewv)

    def _zv(i, carry):
        zbuf[pl.ds(i * 16, 16)] = jnp.zeros((16,), jnp.float32)
        return carry

    lax.fori_loop(0, SEG // 16, _zv, 0)

    def _zc(j, carry):
        pltpu.sync_copy(zbuf, deg_sh.at[pl.ds(s * ROWS_PER_TILE + j * SEG, SEG)])
        return carry

    lax.fori_loop(0, ROWS_PER_TILE // SEG, _zc, 0)
    plsc.subcore_barrier()

    def _body(k, carry):
        scatters = [
            pltpu.async_copy(ewv.at[4 * k + q], deg_sh.at[dstv.at[4 * k + q]],
                             ssem, add=True)
            for q in range(4)
        ]
        for sc in scatters:
            sc.wait()
        return carry

    lax.fori_loop(0, NSEG // 4, _body, 0)
    plsc.subcore_barrier()
    pltpu.sync_copy(
        deg_sh.at[pl.ds(s * ROWS_PER_TILE, ROWS_PER_TILE)],
        out_hbm.at[c, pl.ds(s * ROWS_PER_TILE, ROWS_PER_TILE)],
    )


@functools.partial(
    pl.kernel,
    out_type=jax.ShapeDtypeStruct((2, N_PAD, HID), jnp.float32),
    mesh=_mesh,
    compiler_params=_sc_params,
    scratch_types=[
        pltpu.VMEM((NSEG, SEG), jnp.int32),
        pltpu.VMEM((NSEG, SEG), jnp.int32),
        pltpu.VMEM((NSEG, SEG), jnp.float32),
        pltpu.VMEM((4, SEG, HID), jnp.float32),
        pltpu.VMEM_SHARED((N_PAD, HID), jnp.float32),
        pltpu.SemaphoreType.DMA,
        pltpu.SemaphoreType.DMA,
        pltpu.SemaphoreType.DMA,
        pltpu.SemaphoreType.DMA,
        pltpu.SemaphoreType.DMA,
    ],
)
def _agg_sc(hh_hbm, src_hbm, dst_hbm, ew_hbm, out_hbm, srcv, dstv, ewv,
            rows, agg_sh, gsem0, gsem1, gsem2, gsem3, ssem):
    gsems = (gsem0, gsem1, gsem2, gsem3)
    c = lax.axis_index("c")
    s = lax.axis_index("s")
    w = c * 16 + s
    pltpu.sync_copy(src_hbm.at[w], srcv)
    pltpu.sync_copy(dst_hbm.at[w], dstv)
    pltpu.sync_copy(ew_hbm.at[w], ewv)

    def _zv(i, carry):
        rows[0, i] = jnp.zeros((HID,), jnp.float32)
        return carry

    lax.fori_loop(0, SEG, _zv, 0)

    def _zc(j, carry):
        pltpu.sync_copy(rows.at[0],
                        agg_sh.at[pl.ds(s * ROWS_PER_TILE + j * SEG, SEG)])
        return carry

    lax.fori_loop(0, ROWS_PER_TILE // SEG, _zc, 0)
    plsc.subcore_barrier()

    def _body(k, carry):
        t0 = 4 * k
        gathers = [
            pltpu.async_copy(hh_hbm.at[srcv.at[t0 + q]], rows.at[q], gsems[q])
            for q in range(4)
        ]
        scatters = []
        for q in range(4):
            t = t0 + q
            gathers[q].wait()

            def _blk(bb, carry2, _q=q, _t=t):
                ew16 = ewv[_t, pl.ds(bb * 16, 16)]
                for j in range(16):
                    e = bb * 16 + j
                    rows[_q, e] = rows[_q, e] * ew16[j]
                return carry2

            lax.fori_loop(0, SEG // 16, _blk, 0)
            scatters.append(
                pltpu.async_copy(rows.at[q], agg_sh.at[dstv.at[t]], ssem,
                                 add=True))
        for sc in scatters:
            sc.wait()
        return carry

    lax.fori_loop(0, NSEG // 4, _body, 0)
    plsc.subcore_barrier()
    pltpu.sync_copy(
        agg_sh.at[pl.ds(s * ROWS_PER_TILE, ROWS_PER_TILE)],
        out_hbm.at[c, pl.ds(s * ROWS_PER_TILE, ROWS_PER_TILE)],
    )


def _pre_tc_body(x_ref, w1_ref, degp_ref, hh_ref, dinv_ref):
    deg = degp_ref[0, :N_NODES] + degp_ref[1, :N_NODES] + 1.0
    dinv = jnp.where(deg > 0, lax.rsqrt(deg), 0.0)
    h = jnp.dot(x_ref[...], w1_ref[...], preferred_element_type=jnp.float32)
    hh_ref[:N_NODES, :] = h * dinv[:, None]
    hh_ref[N_NODES:, :] = jnp.zeros((N_PAD - N_NODES, HID), jnp.float32)
    dinv_ref[...] = dinv[:, None]


_pre_tc = pl.pallas_call(
    _pre_tc_body,
    out_shape=(
        jax.ShapeDtypeStruct((N_PAD, HID), jnp.float32),
        jax.ShapeDtypeStruct((N_NODES, 1), jnp.float32),
    ),
)


def _mid_tc_body(aggp_ref, hh1_ref, dinv_ref, b1_ref, u_ref):
    dinv = dinv_ref[...]
    a = aggp_ref[0, :N_NODES, :] + aggp_ref[1, :N_NODES, :] + hh1_ref[:N_NODES, :]
    h1 = jnp.maximum(dinv * a + b1_ref[...][None, :], 0.0)
    u_ref[:N_NODES, :] = h1 * dinv
    u_ref[N_NODES:, :] = jnp.zeros((N_PAD - N_NODES, HID), jnp.float32)


_mid_tc = pl.pallas_call(
    _mid_tc_body,
    out_shape=jax.ShapeDtypeStruct((N_PAD, HID), jnp.float32),
)


def _post_tc_body(aggp_ref, u_ref, dinv_ref, w2_ref, b2_ref, batch_ref,
                  wfc_ref, bfc_ref, out_ref):
    dinv = dinv_ref[...]
    a = aggp_ref[0, :N_NODES, :] + aggp_ref[1, :N_NODES, :] + u_ref[:N_NODES, :]
    aw = jnp.dot(a, w2_ref[...], preferred_element_type=jnp.float32)
    h2 = dinv * aw + b2_ref[...][None, :]
    b = batch_ref[...]
    gids = lax.broadcasted_iota(jnp.int32, (N_NODES, N_GRAPHS), 1)
    m = (b[:, None] == gids).astype(jnp.float32)
    sums = lax.dot_general(m, h2, (((0,), (0,)), ((), ())),
                           preferred_element_type=jnp.float32)
    counts = jnp.sum(m, axis=0)[:, None]
    pooled = sums / jnp.maximum(counts, 1.0)
    out_ref[...] = (
        jnp.dot(pooled, wfc_ref[...], preferred_element_type=jnp.float32)
        + bfc_ref[...][None, :]
    )


_post_tc = pl.pallas_call(
    _post_tc_body,
    out_shape=jax.ShapeDtypeStruct((N_GRAPHS, 1), jnp.float32),
    compiler_params=pltpu.CompilerParams(fuse_transposed_lhs_in_matmul=True),
)


def kernel(x, edge_index, edge_weight, batch, W1, b1, W2, b2, Wfc, bfc):
    pad = EP - N_EDGES
    padidx = N_NODES + (jnp.arange(pad, dtype=jnp.int32) % (N_PAD - N_NODES))
    srcp = jnp.concatenate([edge_index[0], padidx]).reshape(NW, NSEG, SEG)
    dstp = jnp.concatenate([edge_index[1], padidx]).reshape(NW, NSEG, SEG)
    ewp = jnp.concatenate(
        [edge_weight, jnp.zeros((pad,), jnp.float32)]
    ).reshape(NW, NSEG, SEG)

    degp = _deg_sc(dstp, ewp)
    hh1, dinv = _pre_tc(x, W1, degp)
    agg1 = _agg_sc(hh1, srcp, dstp, ewp)
    u = _mid_tc(agg1, hh1, dinv, b1)
    agg2 = _agg_sc(u, srcp, dstp, ewp)
    return _post_tc(agg2, u, dinv, W2, b2, batch, Wfc, bfc)


# speedup vs baseline: 63.3649x; 1.1685x over previous
import functools

import jax
import jax.numpy as jnp
from jax import lax
from jax.experimental import pallas as pl
from jax.experimental.pallas import tpu as pltpu
from jax.experimental.pallas import tpu_sc as plsc

N_NODES = 10000
N_PAD = 10240
N_EDGES = 320000
NW = 32
NSEG = 84
SEG = 128
EP = NW * NSEG * SEG
ROWS_PER_TILE = N_PAD // 16
IN_CH = 128
HID = 16
N_GRAPHS = 64

_mesh = plsc.VectorSubcoreMesh(core_axis_name="c", subcore_axis_name="s")
_sc_params = pltpu.CompilerParams(use_tc_tiling_on_sc=False,
                                  needs_layout_passes=False)


@functools.partial(
    pl.kernel,
    out_type=jax.ShapeDtypeStruct((2, N_PAD), jnp.float32),
    mesh=_mesh,
    compiler_params=_sc_params,
    scratch_types=[
        pltpu.VMEM((NSEG, SEG), jnp.int32),
        pltpu.VMEM((NSEG, SEG), jnp.float32),
        pltpu.VMEM((SEG,), jnp.float32),
        pltpu.VMEM_SHARED((N_PAD,), jnp.float32),
        pltpu.SemaphoreType.DMA,
    ],
)
def _deg_sc(dst_hbm, ew_hbm, out_hbm, dstv, ewv, zbuf, deg_sh, ssem):
    c = lax.axis_index("c")
    s = lax.axis_index("s")
    w = c * 16 + s
    pltpu.sync_copy(dst_hbm.at[w], dstv)
    pltpu.sync_copy(ew_hbm.at[w], ewv)

    def _zv(i, carry):
        zbuf[pl.ds(i * 16, 16)] = jnp.zeros((16,), jnp.float32)
        return carry

    lax.fori_loop(0, SEG // 16, _zv, 0)

    def _zc(j, carry):
        pltpu.sync_copy(zbuf, deg_sh.at[pl.ds(s * ROWS_PER_TILE + j * SEG, SEG)])
        return carry

    lax.fori_loop(0, ROWS_PER_TILE // SEG, _zc, 0)
    plsc.subcore_barrier()

    def _body(k, carry):
        scatters = [
            pltpu.async_copy(ewv.at[4 * k + q], deg_sh.at[dstv.at[4 * k + q]],
                             ssem, add=True)
            for q in range(4)
        ]
        for sc in scatters:
            sc.wait()
        return carry

    lax.fori_loop(0, NSEG // 4, _body, 0)
    plsc.subcore_barrier()
    pltpu.sync_copy(
        deg_sh.at[pl.ds(s * ROWS_PER_TILE, ROWS_PER_TILE)],
        out_hbm.at[c, pl.ds(s * ROWS_PER_TILE, ROWS_PER_TILE)],
    )


@functools.partial(
    pl.kernel,
    out_type=jax.ShapeDtypeStruct((2, N_PAD, HID), jnp.float32),
    mesh=_mesh,
    compiler_params=_sc_params,
    scratch_types=[
        pltpu.VMEM((NSEG, SEG), jnp.int32),
        pltpu.VMEM((NSEG, SEG), jnp.int32),
        pltpu.VMEM((NSEG, SEG), jnp.float32),
        pltpu.VMEM((N_PAD,), jnp.float32),
        pltpu.VMEM((4, SEG, HID), jnp.float32),
        pltpu.VMEM_SHARED((N_PAD, HID), jnp.float32),
        pltpu.SemaphoreType.DMA,
        pltpu.SemaphoreType.DMA,
        pltpu.SemaphoreType.DMA,
        pltpu.SemaphoreType.DMA,
        pltpu.SemaphoreType.DMA,
    ],
)
def _agg_sc(hh_hbm, dinv_hbm, src_hbm, dst_hbm, ew_hbm, out_hbm, srcv, dstv,
            ewv, dv, rows, agg_sh, gsem0, gsem1, gsem2, gsem3, ssem):
    gsems = (gsem0, gsem1, gsem2, gsem3)
    c = lax.axis_index("c")
    s = lax.axis_index("s")
    w = c * 16 + s
    pltpu.sync_copy(src_hbm.at[w], srcv)
    pltpu.sync_copy(dst_hbm.at[w], dstv)
    pltpu.sync_copy(ew_hbm.at[w], ewv)
    pltpu.sync_copy(dinv_hbm, dv)

    def _zv(i, carry):
        rows[0, i] = jnp.zeros((HID,), jnp.float32)
        return carry

    lax.fori_loop(0, SEG, _zv, 0)

    def _zc(j, carry):
        pltpu.sync_copy(rows.at[0],
                        agg_sh.at[pl.ds(s * ROWS_PER_TILE + j * SEG, SEG)])
        return carry

    lax.fori_loop(0, ROWS_PER_TILE // SEG, _zc, 0)
    plsc.subcore_barrier()

    def _body(k, carry):
        t0 = 4 * k
        gathers = [
            pltpu.async_copy(hh_hbm.at[srcv.at[t0 + q]], rows.at[q], gsems[q])
            for q in range(4)
        ]
        scatters = []
        for q in range(4):
            t = t0 + q
            gathers[q].wait()

            def _blk(bb, carry2, _q=q, _t=t):
                ew16 = ewv[_t, pl.ds(bb * 16, 16)]
                s16 = srcv[_t, pl.ds(bb * 16, 16)]
                d16 = dstv[_t, pl.ds(bb * 16, 16)]
                n16 = (plsc.load_gather(dv, [s16]) * ew16
                       * plsc.load_gather(dv, [d16]))
                for j in range(16):
                    e = bb * 16 + j
                    rows[_q, e] = rows[_q, e] * n16[j]
                return carry2

            lax.fori_loop(0, SEG // 16, _blk, 0)
            scatters.append(
                pltpu.async_copy(rows.at[q], agg_sh.at[dstv.at[t]], ssem,
                                 add=True))
        for sc in scatters:
            sc.wait()
        return carry

    lax.fori_loop(0, NSEG // 4, _body, 0)
    plsc.subcore_barrier()
    pltpu.sync_copy(
        agg_sh.at[pl.ds(s * ROWS_PER_TILE, ROWS_PER_TILE)],
        out_hbm.at[c, pl.ds(s * ROWS_PER_TILE, ROWS_PER_TILE)],
    )


NPK = N_PAD // 8


def _kron8(w):
    row = jnp.concatenate([w] * 8, axis=1)
    t = jnp.concatenate([row] * 8, axis=0)
    kr, kc = w.shape
    rb = lax.broadcasted_iota(jnp.int32, (8 * kr, 8 * kc), 0) // kr
    cb = lax.broadcasted_iota(jnp.int32, (8 * kr, 8 * kc), 1) // kc
    return jnp.where(rb == cb, t, 0.0)


def _pre_tc_body(x8_ref, w1_ref, degp_ref, h_ref, dinv_ref):
    deg = degp_ref[0] + degp_ref[1]
    dinv_ref[...] = jnp.where(deg > 0, lax.rsqrt(deg), 0.0)
    h_ref[...] = jnp.dot(x8_ref[...], _kron8(w1_ref[...]),
                         preferred_element_type=jnp.float32)


_pre_tc = pl.pallas_call(
    _pre_tc_body,
    out_shape=(
        jax.ShapeDtypeStruct((NPK, 128), jnp.float32),
        jax.ShapeDtypeStruct((N_PAD // 128, 128), jnp.float32),
    ),
)


def _mid_tc_body(aggp_ref, b1_ref, u_ref):
    b1t = jnp.concatenate([b1_ref[...]] * 8)[None, :]
    u_ref[...] = jnp.maximum(aggp_ref[0] + aggp_ref[1] + b1t, 0.0)


_mid_tc = pl.pallas_call(
    _mid_tc_body,
    out_shape=jax.ShapeDtypeStruct((NPK, 128), jnp.float32),
)


def _post_tc_body(aggp_ref, w2_ref, b2_ref, batch8_ref, wfc_ref, bfc_ref,
                  out_ref):
    ap = aggp_ref[0] + aggp_ref[1]
    b2t = jnp.concatenate([b2_ref[...]] * 8)
    h2p = (jnp.dot(ap, _kron8(w2_ref[...]),
                   preferred_element_type=jnp.float32) + b2t[None, :])
    gids = lax.broadcasted_iota(jnp.int32, (NPK, N_GRAPHS), 1)
    sums = jnp.zeros((N_GRAPHS, HID), jnp.float32)
    counts = jnp.zeros((N_GRAPHS,), jnp.float32)
    for a in range(8):
        ba = batch8_ref[a]
        ma = (ba[:, None] == gids).astype(jnp.float32)
        h2a = h2p[:, HID * a:HID * (a + 1)]
        sums = sums + lax.dot_general(ma, h2a, (((0,), (0,)), ((), ())),
                                      preferred_element_type=jnp.float32)
        counts = counts + jnp.sum(ma, axis=0)
    pooled = sums / jnp.maximum(counts[:, None], 1.0)
    out_ref[...] = (
        jnp.dot(pooled, wfc_ref[...], preferred_element_type=jnp.float32)
        + bfc_ref[...][None, :]
    )


_post_tc = pl.pallas_call(
    _post_tc_body,
    out_shape=jax.ShapeDtypeStruct((N_GRAPHS, 1), jnp.float32),
    compiler_params=pltpu.CompilerParams(fuse_transposed_lhs_in_matmul=True),
)


def kernel(x, edge_index, edge_weight, batch, W1, b1, W2, b2, Wfc, bfc):
    selfidx = jnp.arange(N_PAD, dtype=jnp.int32)
    tail = EP - N_EDGES - N_PAD
    padidx = N_NODES + (jnp.arange(tail, dtype=jnp.int32) % (N_PAD - N_NODES))
    srcp = jnp.concatenate(
        [edge_index[0], selfidx, padidx]).reshape(NW, NSEG, SEG)
    dstp = jnp.concatenate(
        [edge_index[1], selfidx, padidx]).reshape(NW, NSEG, SEG)
    ewp = jnp.concatenate(
        [edge_weight, jnp.ones((N_PAD,), jnp.float32),
         jnp.zeros((tail,), jnp.float32)]).reshape(NW, NSEG, SEG)
    x8 = jnp.concatenate(
        [x, jnp.zeros((N_PAD - N_NODES, IN_CH), jnp.float32)]
    ).reshape(NPK, 8 * IN_CH)

    degp = _deg_sc(dstp, ewp)
    hp, dinv2d = _pre_tc(x8, W1, degp.reshape(2, N_PAD // 128, 128))
    dinv = dinv2d.reshape(N_PAD)
    agg1 = _agg_sc(hp.reshape(N_PAD, HID), dinv, srcp, dstp, ewp)
    up = _mid_tc(agg1.reshape(2, NPK, 128), b1)
    agg2 = _agg_sc(up.reshape(N_PAD, HID), dinv, srcp, dstp, ewp)
    batch8 = jnp.concatenate(
        [batch, jnp.full((N_PAD - N_NODES,), -1, jnp.int32)]
    ).reshape(NPK, 8).transpose()
    return _post_tc(agg2.reshape(2, NPK, 128), W2, b2, batch8, Wfc, bfc)
